# baseline (speedup 1.0000x reference)
import functools
import math

import numpy as np
import jax
import jax.numpy as jnp
from jax import lax
from jax.experimental import pallas as pl
from jax.experimental.pallas import tpu as pltpu
from jax.experimental.pallas import tpu_sc as plsc

ROW = 16
RSIZE = ROW * ROW
OSIZE = (ROW * (ROW + 1)) // 2
NWORKERS = 32
PERIOD = 2
TABLE = PERIOD * OSIZE
NVREG = TABLE // 16


def _tables():
    iu0, iu1 = np.triu_indices(ROW)
    flat = (iu0 * ROW + iu1).astype(np.int32)
    off = np.concatenate([flat, flat + RSIZE])
    scl = np.where(iu0 == iu1, 1.0, math.sqrt(2.0)).astype(np.float32)
    scl = np.concatenate([scl, scl])
    return off, scl


_OFF_NP, _SCL_NP = _tables()


@functools.cache
def _build(n_rows: int):
    rows_w = n_rows // NWORKERS
    blk_r = 100
    while rows_w % blk_r or blk_r % PERIOD:
        blk_r -= 1
    nblk = rows_w // blk_r
    blk_in = blk_r * RSIZE
    blk_out = blk_r * OSIZE
    pairs = blk_r // PERIOD

    mesh = plsc.VectorSubcoreMesh(core_axis_name="c", subcore_axis_name="s")

    @functools.partial(
        pl.kernel,
        mesh=mesh,
        out_type=jax.ShapeDtypeStruct((n_rows * OSIZE,), jnp.float32),
        scratch_types=[
            pltpu.VMEM((blk_in,), jnp.float32),
            pltpu.VMEM((blk_out,), jnp.float32),
            pltpu.VMEM((TABLE,), jnp.int32),
            pltpu.VMEM((TABLE,), jnp.float32),
        ],
        compiler_params=pltpu.CompilerParams(needs_layout_passes=False),
    )
    def run(x_hbm, off_hbm, scl_hbm, out_hbm, in_v, out_v, off_v, scl_v):
        cid = lax.axis_index("c")
        sid = lax.axis_index("s")
        wid = sid * 2 + cid
        pltpu.sync_copy(off_hbm, off_v)
        pltpu.sync_copy(scl_hbm, scl_v)

        def blk_body(g, carry):
            blk_id = wid * nblk + g
            pltpu.sync_copy(x_hbm.at[pl.ds(blk_id * blk_in, blk_in)], in_v)

            def pair_body(p, c2):
                base = p * (PERIOD * RSIZE)
                obase = p * TABLE
                for v in range(NVREG):
                    idx = off_v[pl.ds(v * 16, 16)] + base
                    vals = plsc.load_gather(in_v, [idx])
                    res = jnp.abs(vals) * scl_v[pl.ds(v * 16, 16)]
                    out_v[pl.ds(obase + v * 16, 16)] = res
                return c2

            lax.fori_loop(0, pairs, pair_body, 0)
            pltpu.sync_copy(out_v, out_hbm.at[pl.ds(blk_id * blk_out, blk_out)])
            return carry

        lax.fori_loop(0, nblk, blk_body, 0)

    return run


def kernel(input_st):
    shape = input_st.shape
    n_rows = int(np.prod(shape[:-2]))
    x = input_st.reshape(n_rows * RSIZE)
    off = jnp.asarray(_OFF_NP)
    scl = jnp.asarray(_SCL_NP)
    out = _build(n_rows)(x, off, scl)
    return out.reshape(*shape[:-2], OSIZE, 1)


# speedup vs baseline: 1.2579x; 1.2579x over previous
import functools
import math

import numpy as np
import jax
import jax.numpy as jnp
from jax import lax
from jax.experimental import pallas as pl
from jax.experimental.pallas import tpu as pltpu
from jax.experimental.pallas import tpu_sc as plsc

ROW = 16
RSIZE = ROW * ROW
OSIZE = (ROW * (ROW + 1)) // 2
NWORKERS = 32
PERIOD = 2
TABLE = PERIOD * OSIZE
NVREG = TABLE // 16


def _tables():
    iu0, iu1 = np.triu_indices(ROW)
    flat = (iu0 * ROW + iu1).astype(np.int32)
    off = np.concatenate([flat, flat + RSIZE])
    scl = np.where(iu0 == iu1, 1.0, math.sqrt(2.0)).astype(np.float32)
    scl = np.concatenate([scl, scl])
    return off, scl


_OFF_NP, _SCL_NP = _tables()


@functools.cache
def _build(n_rows: int):
    rows_w = n_rows // NWORKERS
    blk_r = 100
    while rows_w % blk_r or blk_r % PERIOD:
        blk_r -= 1
    nblk = rows_w // blk_r
    blk_in = blk_r * RSIZE
    blk_out = blk_r * OSIZE
    pairs = blk_r // PERIOD

    mesh = plsc.VectorSubcoreMesh(core_axis_name="c", subcore_axis_name="s")

    @functools.partial(
        pl.kernel,
        mesh=mesh,
        out_type=jax.ShapeDtypeStruct((n_rows * OSIZE,), jnp.float32),
        scratch_types=[
            pltpu.VMEM((blk_in,), jnp.float32),
            pltpu.VMEM((blk_in,), jnp.float32),
            pltpu.VMEM((blk_out,), jnp.float32),
            pltpu.VMEM((blk_out,), jnp.float32),
            pltpu.VMEM((TABLE,), jnp.int32),
            pltpu.VMEM((TABLE,), jnp.float32),
            pltpu.SemaphoreType.DMA,
            pltpu.SemaphoreType.DMA,
            pltpu.SemaphoreType.DMA,
            pltpu.SemaphoreType.DMA,
        ],
        compiler_params=pltpu.CompilerParams(needs_layout_passes=False),
    )
    def run(x_hbm, off_hbm, scl_hbm, out_hbm,
            in_v0, in_v1, out_v0, out_v1, off_v, scl_v, si0, si1, so0, so1):
        cid = lax.axis_index("c")
        sid = lax.axis_index("s")
        wid = sid * 2 + cid
        pltpu.sync_copy(off_hbm, off_v)
        pltpu.sync_copy(scl_hbm, scl_v)
        offs = [off_v[pl.ds(v * 16, 16)] for v in range(NVREG)]
        scls = [scl_v[pl.ds(v * 16, 16)] for v in range(NVREG)]
        in_bufs = (in_v0, in_v1)
        out_bufs = (out_v0, out_v1)
        isems = (si0, si1)
        osems = (so0, so1)

        def in_copy(g, b):
            return pltpu.make_async_copy(
                x_hbm.at[pl.ds((wid * nblk + g) * blk_in, blk_in)],
                in_bufs[b], isems[b])

        def out_copy(g, b):
            return pltpu.make_async_copy(
                out_bufs[b],
                out_hbm.at[pl.ds((wid * nblk + g) * blk_out, blk_out)],
                osems[b])

        in_copy(0, 0).start()
        in_copy(1, 1).start()

        def compute(b):
            in_v = in_bufs[b]
            out_v = out_bufs[b]

            @plsc.parallel_loop(0, pairs, step=1, unroll=2)
            def _(p):
                base = p * (PERIOD * RSIZE)
                obase = p * TABLE
                for v in range(NVREG):
                    vals = plsc.load_gather(in_v, [offs[v] + base])
                    out_v[pl.ds(obase + v * 16, 16)] = jnp.abs(vals) * scls[v]

        def super_body(t, carry):
            for b in range(2):
                g = t * 2 + b
                in_copy(g, b).wait()

                @pl.when(g >= 2)
                def _():
                    out_copy(g - 2, b).wait()

                compute(b)
                out_copy(g, b).start()

                @pl.when(g + 2 < nblk)
                def _():
                    in_copy(g + 2, b).start()
            return carry

        lax.fori_loop(0, nblk // 2, super_body, 0)
        out_copy(nblk - 2, 0).wait()
        out_copy(nblk - 1, 1).wait()

    return run


def kernel(input_st):
    shape = input_st.shape
    n_rows = int(np.prod(shape[:-2]))
    x = input_st.reshape(n_rows * RSIZE)
    off = jnp.asarray(_OFF_NP)
    scl = jnp.asarray(_SCL_NP)
    out = _build(n_rows)(x, off, scl)
    return out.reshape(*shape[:-2], OSIZE, 1)


# speedup vs baseline: 8.6367x; 6.8658x over previous
import functools
import math

import numpy as np
import jax
import jax.numpy as jnp
from jax import lax
from jax.experimental import pallas as pl
from jax.experimental.pallas import tpu as pltpu
from jax.experimental.pallas import tpu_sc as plsc

S2 = math.sqrt(2.0)
NWORKERS = 32
NB, NF, NT, ROW = 32, 10, 500, 16
NBF = NB * NF
BF_PER_W = NBF // NWORKERS
OSIZE = (ROW * (ROW + 1)) // 2
K0 = [ROW * i - i * (i - 1) // 2 for i in range(ROW + 1)]
NVC = (NT + 15) // 16
TAIL = NT - 16 * (NVC - 1)

_IOTA = np.arange(16, dtype=np.int32)
_TMASK = (_IOTA < TAIL)

mesh = plsc.VectorSubcoreMesh(core_axis_name="c", subcore_axis_name="s")


@functools.partial(
    pl.kernel,
    mesh=mesh,
    out_type=jax.ShapeDtypeStruct((NBF * OSIZE, 1, NT), jnp.float32),
    scratch_types=[
        pltpu.VMEM((32, NT), jnp.float32),
        pltpu.VMEM((32, NT), jnp.float32),
        pltpu.VMEM((OSIZE, NT), jnp.float32),
        pltpu.SemaphoreType.DMA,
        pltpu.SemaphoreType.DMA,
        pltpu.SemaphoreType.DMA,
        pltpu.SemaphoreType.DMA,
    ],
    compiler_params=pltpu.CompilerParams(
        needs_layout_passes=False, use_tc_tiling_on_sc=True),
)
def _run(x_hbm, out_hbm, in0, in1, obuf, si0, si1, sd1, sd2):
    wid = lax.axis_index("s") * 2 + lax.axis_index("c")
    bf0 = wid * BF_PER_W
    iota = lax.broadcasted_iota(jnp.int32, (16,), 0)
    tmask = iota < TAIL
    in_bufs = (in0, in1)
    isems = (si0, si1)

    def in_copy(bf, c, b):
        return pltpu.make_async_copy(
            x_hbm.at[bf, pl.ds(32 * c, 32), :], in_bufs[b], isems[b])

    def d1_copy(bf):
        return pltpu.make_async_copy(
            obuf.at[pl.ds(0, 64), :],
            out_hbm.at[pl.ds(bf * OSIZE, 64), 0, :], sd1)

    def d2_copy(bf):
        return pltpu.make_async_copy(
            obuf.at[pl.ds(64, 72), :],
            out_hbm.at[pl.ds(bf * OSIZE + 64, 72), 0, :], sd2)

    def compute(c, b):
        ib = in_bufs[b]
        for di in range(2):
            i = 2 * c + di
            rloc0 = di * ROW + i
            k0 = K0[i]

            def row_body(jj, carry, rloc0=rloc0, k0=k0):
                rvec = jnp.full((16,), rloc0 + jj, jnp.int32)
                kvec = jnp.full((16,), k0 + jj, jnp.int32)
                scale = jnp.where(jj == 0, 1.0, S2)
                for v in range(NVC):
                    col = iota + (16 * v)
                    m = None if v < NVC - 1 else tmask
                    vals = plsc.load_gather(ib, [rvec, col], mask=m)
                    y = jnp.abs(vals) * scale
                    plsc.store_scatter(obuf, [kvec, col], y, mask=m)
                return carry

            lax.fori_loop(0, ROW - i, row_body, 0)

    def bf_body(t, carry):
        bf = bf0 + t
        for c in range(8):
            b = c % 2
            in_copy(bf, c, b).wait()
            if c < 7:
                in_copy(bf, c + 1, 1 - b).start()
            else:
                @pl.when(t < BF_PER_W - 1)
                def _():
                    in_copy(bf + 1, 0, 1 - b).start()

            if c == 0:
                @pl.when(t > 0)
                def _(bf=bf):
                    d1_copy(bf - 1).wait()
            if c == 2:
                @pl.when(t > 0)
                def _(bf=bf):
                    d2_copy(bf - 1).wait()

            compute(c, b)
            if c == 2:
                d1_copy(bf).start()
        d2_copy(bf).start()
        return carry

    in_copy(bf0, 0, 0).start()
    lax.fori_loop(0, BF_PER_W, bf_body, 0)
    d1_copy(bf0 + BF_PER_W - 1).wait()
    d2_copy(bf0 + BF_PER_W - 1).wait()


def kernel(input_st):
    xv = input_st.transpose(0, 1, 3, 4, 2).reshape(NBF, ROW * ROW, NT)
    out = _run(xv)
    return out.reshape(NB, NF, OSIZE, 1, NT).transpose(0, 1, 4, 2, 3)


# speedup vs baseline: 10.5714x; 1.2240x over previous
import functools
import math

import numpy as np
import jax
import jax.numpy as jnp
from jax import lax
from jax.experimental import pallas as pl
from jax.experimental.pallas import tpu as pltpu
from jax.experimental.pallas import tpu_sc as plsc

S2 = math.sqrt(2.0)
NWORKERS = 32
NB, NF, NT, ROW = 32, 10, 500, 16
NBF = NB * NF
BF_PER_W = NBF // NWORKERS
OSIZE = (ROW * (ROW + 1)) // 2
K0 = [ROW * i - i * (i - 1) // 2 for i in range(ROW + 1)]
NVC = (NT + 15) // 16
TAIL = NT - 16 * (NVC - 1)

_IOTA = np.arange(16, dtype=np.int32)
_TMASK = (_IOTA < TAIL)

mesh = plsc.VectorSubcoreMesh(core_axis_name="c", subcore_axis_name="s")


@functools.partial(
    pl.kernel,
    mesh=mesh,
    out_type=jax.ShapeDtypeStruct((NBF * OSIZE, 1, NT), jnp.float32),
    scratch_types=[
        pltpu.VMEM((32, NT), jnp.float32),
        pltpu.VMEM((32, NT), jnp.float32),
        pltpu.VMEM((OSIZE, NT), jnp.float32),
        pltpu.SemaphoreType.DMA,
        pltpu.SemaphoreType.DMA,
        pltpu.SemaphoreType.DMA,
        pltpu.SemaphoreType.DMA,
    ],
    compiler_params=pltpu.CompilerParams(
        needs_layout_passes=False, use_tc_tiling_on_sc=True),
)
def _run(x_hbm, out_hbm, in0, in1, obuf, si0, si1, sd1, sd2):
    wid = lax.axis_index("s") * 2 + lax.axis_index("c")
    bf0 = wid * BF_PER_W
    iota = lax.broadcasted_iota(jnp.int32, (16,), 0)
    tmask = iota < TAIL
    in_bufs = (in0, in1)
    isems = (si0, si1)

    def in_copy(bf, c, b):
        return pltpu.make_async_copy(
            x_hbm.at[bf, pl.ds(32 * c, 32), :], in_bufs[b], isems[b])

    def d1_copy(bf):
        return pltpu.make_async_copy(
            obuf.at[pl.ds(0, 64), :],
            out_hbm.at[pl.ds(bf * OSIZE, 64), 0, :], sd1)

    def d2_copy(bf):
        return pltpu.make_async_copy(
            obuf.at[pl.ds(64, 72), :],
            out_hbm.at[pl.ds(bf * OSIZE + 64, 72), 0, :], sd2)

    def compute(c, b):
        ib = in_bufs[b]
        for di in range(2):
            i = 2 * c + di
            rloc0 = di * ROW + i
            k0 = K0[i]

            def row_body(jj, carry, rloc0=rloc0, k0=k0):
                r = rloc0 + jj
                k = k0 + jj
                scale = jnp.where(jj == 0, 1.0, S2)
                for v in range(NVC - 1):
                    vals = ib[r, pl.ds(16 * v, 16)]
                    obuf[k, pl.ds(16 * v, 16)] = jnp.abs(vals) * scale
                rvec = jnp.full((16,), r, jnp.int32)
                kvec = jnp.full((16,), k, jnp.int32)
                col = iota + 16 * (NVC - 1)
                vals = plsc.load_gather(ib, [rvec, col], mask=tmask)
                plsc.store_scatter(
                    obuf, [kvec, col], jnp.abs(vals) * scale, mask=tmask)
                return carry

            lax.fori_loop(0, ROW - i, row_body, 0)

    def bf_body(t, carry):
        bf = bf0 + t
        for c in range(8):
            b = c % 2
            in_copy(bf, c, b).wait()
            if c < 7:
                in_copy(bf, c + 1, 1 - b).start()
            else:
                @pl.when(t < BF_PER_W - 1)
                def _():
                    in_copy(bf + 1, 0, 1 - b).start()

            if c == 0:
                @pl.when(t > 0)
                def _(bf=bf):
                    d1_copy(bf - 1).wait()
            if c == 2:
                @pl.when(t > 0)
                def _(bf=bf):
                    d2_copy(bf - 1).wait()

            compute(c, b)
            if c == 2:
                d1_copy(bf).start()
        d2_copy(bf).start()
        return carry

    in_copy(bf0, 0, 0).start()
    lax.fori_loop(0, BF_PER_W, bf_body, 0)
    d1_copy(bf0 + BF_PER_W - 1).wait()
    d2_copy(bf0 + BF_PER_W - 1).wait()


def kernel(input_st):
    xv = input_st.transpose(0, 1, 3, 4, 2).reshape(NBF, ROW * ROW, NT)
    out = _run(xv)
    return out.reshape(NB, NF, OSIZE, 1, NT).transpose(0, 1, 4, 2, 3)
